# baseline (speedup 1.0000x reference)
import functools

import jax
import jax.numpy as jnp
from jax import lax
from jax.experimental import pallas as pl
from jax.experimental.pallas import tpu as pltpu
from jax.experimental.pallas import tpu_sc as plsc

_T, _D, _F, _E = 64, 2048, 4096, 8
_LANES = 16
_BF = 256


def _route_sc_body(lg_hbm, out_hbm, lt, ot):
    wid = lax.axis_index("c") * 16 + lax.axis_index("s")

    @pl.when(wid < _T // _LANES)
    def _():
        pltpu.sync_copy(lg_hbm.at[wid], lt)
        v = [lt[pl.ds(e * _LANES, _LANES)] for e in range(_E)]
        m1 = v[0]
        for e in range(1, _E):
            m1 = jnp.maximum(m1, v[e])
        sent = jnp.full((_LANES,), _E, jnp.int32)
        idx1 = sent
        for e in range(_E):
            ev = jnp.full((_LANES,), e, jnp.int32)
            take = (v[e] == m1) & (idx1 == sent)
            idx1 = jnp.where(take, ev, idx1)
        neg = jnp.full((_LANES,), -jnp.inf, jnp.float32)
        m2 = neg
        for e in range(_E):
            ev = jnp.full((_LANES,), e, jnp.int32)
            m2 = jnp.maximum(m2, jnp.where(idx1 == ev, neg, v[e]))
        idx2 = sent
        for e in range(_E):
            ev = jnp.full((_LANES,), e, jnp.int32)
            take = (v[e] == m2) & (idx2 == sent) & (idx1 != ev)
            idx2 = jnp.where(take, ev, idx2)
        r2 = jnp.exp(m2 - m1)
        tot = r2 + 1.0
        w1v = 1.0 / tot
        w2v = r2 / tot
        zero = jnp.zeros((_LANES,), jnp.float32)
        for e in range(_E):
            ev = jnp.full((_LANES,), e, jnp.int32)
            ot[pl.ds(e * _LANES, _LANES)] = jnp.where(
                idx1 == ev, w1v, jnp.where(idx2 == ev, w2v, zero))
        pltpu.sync_copy(ot, out_hbm.at[wid])


def _route_sc(logits_chunks):
    nchunks = _T // _LANES
    row = _E * _LANES
    mesh = plsc.VectorSubcoreMesh(core_axis_name="c", subcore_axis_name="s")
    k = functools.partial(
        pl.kernel,
        mesh=mesh,
        out_type=jax.ShapeDtypeStruct((nchunks, row), jnp.float32),
        scratch_types=[
            pltpu.VMEM((row,), jnp.float32),
            pltpu.VMEM((row,), jnp.float32),
        ],
    )(_route_sc_body)
    return k(logits_chunks)


def _moe_tc_body(x_ref, rt_ref, w1_ref, w3_ref, w2_ref, out_ref):
    e = pl.program_id(0)
    f = pl.program_id(1)

    @pl.when((e == 0) & (f == 0))
    def _():
        out_ref[...] = jnp.zeros_like(out_ref)

    xb = x_ref[...]
    dn = (((1,), (1,)), ((), ()))
    h1 = lax.dot_general(xb, w1_ref[0], dn, preferred_element_type=jnp.float32)
    h3 = lax.dot_general(xb, w3_ref[0], dn, preferred_element_type=jnp.float32)
    h = h1 * lax.logistic(h1) * h3
    h = h * rt_ref[0]
    out_ref[...] += lax.dot_general(h, w2_ref[0], dn,
                                    preferred_element_type=jnp.float32)


def _moe_tc(x, route_exp, w1_weight, w3_weight, w2_weight):
    grid = (_E, _F // _BF)
    return pl.pallas_call(
        _moe_tc_body,
        grid=grid,
        in_specs=[
            pl.BlockSpec((_T, _D), lambda e, f: (0, 0)),
            pl.BlockSpec((1, _T, 1), lambda e, f: (e, 0, 0)),
            pl.BlockSpec((1, _BF, _D), lambda e, f: (e, f, 0)),
            pl.BlockSpec((1, _BF, _D), lambda e, f: (e, f, 0)),
            pl.BlockSpec((1, _D, _BF), lambda e, f: (e, 0, f)),
        ],
        out_specs=pl.BlockSpec((_T, _D), lambda e, f: (0, 0)),
        out_shape=jax.ShapeDtypeStruct((_T, _D), jnp.float32),
        compiler_params=pltpu.CompilerParams(
            dimension_semantics=("arbitrary", "arbitrary"),
        ),
    )(x, route_exp, w1_weight, w3_weight, w2_weight)


def kernel(x, router_logits, w1_weight, w3_weight, w2_weight):
    nchunks = _T // _LANES
    lg = (router_logits.astype(jnp.float32)
          .reshape(nchunks, _LANES, _E).transpose(0, 2, 1).reshape(nchunks, -1))
    rw = _route_sc(lg)
    route_t = rw.reshape(nchunks, _E, _LANES).transpose(1, 0, 2).reshape(_E, _T)
    route_exp = route_t.astype(x.dtype).reshape(_E, _T, 1)
    return _moe_tc(x, route_exp, w1_weight, w3_weight, w2_weight)


# speedup vs baseline: 1.1216x; 1.1216x over previous
import functools

import jax
import jax.numpy as jnp
from jax import lax
from jax.experimental import pallas as pl
from jax.experimental.pallas import tpu as pltpu
from jax.experimental.pallas import tpu_sc as plsc

_T, _D, _F, _E = 64, 2048, 4096, 8
_LANES = 16
_BF = 512


def _route_sc_body(lg_hbm, out_hbm, lt, ot):
    wid = lax.axis_index("c") * 16 + lax.axis_index("s")

    @pl.when(wid < _T // _LANES)
    def _():
        pltpu.sync_copy(lg_hbm.at[wid], lt)
        v = [lt[pl.ds(e * _LANES, _LANES)] for e in range(_E)]
        m1 = v[0]
        for e in range(1, _E):
            m1 = jnp.maximum(m1, v[e])
        sent = jnp.full((_LANES,), _E, jnp.int32)
        idx1 = sent
        for e in range(_E):
            ev = jnp.full((_LANES,), e, jnp.int32)
            take = (v[e] == m1) & (idx1 == sent)
            idx1 = jnp.where(take, ev, idx1)
        neg = jnp.full((_LANES,), -jnp.inf, jnp.float32)
        m2 = neg
        for e in range(_E):
            ev = jnp.full((_LANES,), e, jnp.int32)
            m2 = jnp.maximum(m2, jnp.where(idx1 == ev, neg, v[e]))
        idx2 = sent
        for e in range(_E):
            ev = jnp.full((_LANES,), e, jnp.int32)
            take = (v[e] == m2) & (idx2 == sent) & (idx1 != ev)
            idx2 = jnp.where(take, ev, idx2)
        r2 = jnp.exp(m2 - m1)
        tot = r2 + 1.0
        w1v = 1.0 / tot
        w2v = r2 / tot
        zero = jnp.zeros((_LANES,), jnp.float32)
        for e in range(_E):
            ev = jnp.full((_LANES,), e, jnp.int32)
            ot[pl.ds(e * _LANES, _LANES)] = jnp.where(
                idx1 == ev, w1v, jnp.where(idx2 == ev, w2v, zero))
        pltpu.sync_copy(ot, out_hbm.at[wid])


def _route_sc(logits_chunks):
    nchunks = _T // _LANES
    row = _E * _LANES
    mesh = plsc.VectorSubcoreMesh(core_axis_name="c", subcore_axis_name="s")
    k = functools.partial(
        pl.kernel,
        mesh=mesh,
        out_type=jax.ShapeDtypeStruct((nchunks, row), jnp.float32),
        scratch_types=[
            pltpu.VMEM((row,), jnp.float32),
            pltpu.VMEM((row,), jnp.float32),
        ],
    )(_route_sc_body)
    return k(logits_chunks)


def _moe_tc_body(x_ref, rt_ref, w1_ref, w3_ref, w2_ref, out_ref):
    e = pl.program_id(0)
    f = pl.program_id(1)

    @pl.when((e == 0) & (f == 0))
    def _():
        out_ref[...] = jnp.zeros_like(out_ref)

    xb = x_ref[...]
    dn = (((1,), (1,)), ((), ()))
    h1 = lax.dot_general(xb, w1_ref[0], dn, preferred_element_type=jnp.float32)
    h3 = lax.dot_general(xb, w3_ref[0], dn, preferred_element_type=jnp.float32)
    h = h1 * lax.logistic(h1) * h3
    h = h * rt_ref[0]
    out_ref[...] += lax.dot_general(h, w2_ref[0], dn,
                                    preferred_element_type=jnp.float32)


def _moe_tc(x, route_exp, w1_weight, w3_weight, w2_weight):
    grid = (_E, _F // _BF)
    return pl.pallas_call(
        _moe_tc_body,
        grid=grid,
        in_specs=[
            pl.BlockSpec((_T, _D), lambda e, f: (0, 0)),
            pl.BlockSpec((1, _T, 1), lambda e, f: (e, 0, 0)),
            pl.BlockSpec((1, _BF, _D), lambda e, f: (e, f, 0)),
            pl.BlockSpec((1, _BF, _D), lambda e, f: (e, f, 0)),
            pl.BlockSpec((1, _D, _BF), lambda e, f: (e, 0, f)),
        ],
        out_specs=pl.BlockSpec((_T, _D), lambda e, f: (0, 0)),
        out_shape=jax.ShapeDtypeStruct((_T, _D), jnp.float32),
        compiler_params=pltpu.CompilerParams(
            dimension_semantics=("arbitrary", "arbitrary"),
        ),
    )(x, route_exp, w1_weight, w3_weight, w2_weight)


def kernel(x, router_logits, w1_weight, w3_weight, w2_weight):
    nchunks = _T // _LANES
    lg = (router_logits.astype(jnp.float32)
          .reshape(nchunks, _LANES, _E).transpose(0, 2, 1).reshape(nchunks, -1))
    rw = _route_sc(lg)
    route_t = rw.reshape(nchunks, _E, _LANES).transpose(1, 0, 2).reshape(_E, _T)
    route_exp = route_t.astype(x.dtype).reshape(_E, _T, 1)
    return _moe_tc(x, route_exp, w1_weight, w3_weight, w2_weight)


# speedup vs baseline: 1.1316x; 1.0090x over previous
import functools

import jax
import jax.numpy as jnp
from jax import lax
from jax.experimental import pallas as pl
from jax.experimental.pallas import tpu as pltpu
from jax.experimental.pallas import tpu_sc as plsc

_T, _D, _F, _E = 64, 2048, 4096, 8
_LANES = 16
_BF = 512


def _route_sc_body(lg_hbm, out_hbm, lt, ot):
    wid = lax.axis_index("c") * 16 + lax.axis_index("s")

    @pl.when(wid < _T // _LANES)
    def _():
        pltpu.sync_copy(lg_hbm.at[wid], lt)
        v = [lt[pl.ds(e * _LANES, _LANES)] for e in range(_E)]
        m1 = v[0]
        for e in range(1, _E):
            m1 = jnp.maximum(m1, v[e])
        sent = jnp.full((_LANES,), _E, jnp.int32)
        idx1 = sent
        for e in range(_E):
            ev = jnp.full((_LANES,), e, jnp.int32)
            take = (v[e] == m1) & (idx1 == sent)
            idx1 = jnp.where(take, ev, idx1)
        neg = jnp.full((_LANES,), -jnp.inf, jnp.float32)
        m2 = neg
        for e in range(_E):
            ev = jnp.full((_LANES,), e, jnp.int32)
            m2 = jnp.maximum(m2, jnp.where(idx1 == ev, neg, v[e]))
        idx2 = sent
        for e in range(_E):
            ev = jnp.full((_LANES,), e, jnp.int32)
            take = (v[e] == m2) & (idx2 == sent) & (idx1 != ev)
            idx2 = jnp.where(take, ev, idx2)
        r2 = jnp.exp(m2 - m1)
        tot = r2 + 1.0
        w1v = 1.0 / tot
        w2v = r2 / tot
        zero = jnp.zeros((_LANES,), jnp.float32)
        for e in range(_E):
            ev = jnp.full((_LANES,), e, jnp.int32)
            ot[pl.ds(e * _LANES, _LANES)] = jnp.where(
                idx1 == ev, w1v, jnp.where(idx2 == ev, w2v, zero))
        for e in range(_E):
            pltpu.sync_copy(ot.at[pl.ds(e * _LANES, _LANES)],
                            out_hbm.at[e, 0, pl.ds(wid * _LANES, _LANES)])


def _route_sc(logits_chunks):
    row = _E * _LANES
    mesh = plsc.VectorSubcoreMesh(core_axis_name="c", subcore_axis_name="s")
    k = functools.partial(
        pl.kernel,
        mesh=mesh,
        out_type=jax.ShapeDtypeStruct((_E, 1, _T), jnp.float32),
        scratch_types=[
            pltpu.VMEM((row,), jnp.float32),
            pltpu.VMEM((row,), jnp.float32),
        ],
    )(_route_sc_body)
    return k(logits_chunks)


def _moe_tc_body(x_ref, rt_ref, w1_ref, w3_ref, w2_ref, out_ref):
    e = pl.program_id(0)
    f = pl.program_id(1)

    @pl.when((e == 0) & (f == 0))
    def _():
        out_ref[...] = jnp.zeros_like(out_ref)

    xb = x_ref[...]
    dn = (((1,), (1,)), ((), ()))
    h1 = lax.dot_general(xb, w1_ref[0], dn, preferred_element_type=jnp.float32)
    h3 = lax.dot_general(xb, w3_ref[0], dn, preferred_element_type=jnp.float32)
    h = h1 * lax.logistic(h1) * h3
    eye = (lax.broadcasted_iota(jnp.int32, (_T, _T), 0)
           == lax.broadcasted_iota(jnp.int32, (_T, _T), 1)).astype(jnp.float32)
    scale = lax.dot_general(eye, rt_ref[0], dn,
                            preferred_element_type=jnp.float32)
    h = h * scale
    out_ref[...] += lax.dot_general(h, w2_ref[0], dn,
                                    preferred_element_type=jnp.float32)


def _moe_tc(x, route_exp, w1_weight, w3_weight, w2_weight):
    grid = (_E, _F // _BF)
    return pl.pallas_call(
        _moe_tc_body,
        grid=grid,
        in_specs=[
            pl.BlockSpec((_T, _D), lambda e, f: (0, 0)),
            pl.BlockSpec((1, 1, _T), lambda e, f: (e, 0, 0)),
            pl.BlockSpec((1, _BF, _D), lambda e, f: (e, f, 0)),
            pl.BlockSpec((1, _BF, _D), lambda e, f: (e, f, 0)),
            pl.BlockSpec((1, _D, _BF), lambda e, f: (e, 0, f)),
        ],
        out_specs=pl.BlockSpec((_T, _D), lambda e, f: (0, 0)),
        out_shape=jax.ShapeDtypeStruct((_T, _D), jnp.float32),
        compiler_params=pltpu.CompilerParams(
            dimension_semantics=("arbitrary", "arbitrary"),
        ),
    )(x, route_exp, w1_weight, w3_weight, w2_weight)


def kernel(x, router_logits, w1_weight, w3_weight, w2_weight):
    nchunks = _T // _LANES
    lg = (router_logits.astype(jnp.float32)
          .reshape(nchunks, _LANES, _E).transpose(0, 2, 1).reshape(nchunks, -1))
    route = _route_sc(lg)
    return _moe_tc(x, route, w1_weight, w3_weight, w2_weight)


# speedup vs baseline: 1.1406x; 1.0080x over previous
import functools

import jax
import jax.numpy as jnp
from jax import lax
from jax.experimental import pallas as pl
from jax.experimental.pallas import tpu as pltpu
from jax.experimental.pallas import tpu_sc as plsc

_T, _D, _F, _E = 64, 2048, 4096, 8
_LANES = 16
_BF = 512


def _route_sc_body(lg_hbm, out_hbm, lt, ot):
    wid = lax.axis_index("s")

    @pl.when(wid < _T // _LANES)
    def _():
        pltpu.sync_copy(lg_hbm.at[wid], lt)
        v = [lt[pl.ds(e * _LANES, _LANES)] for e in range(_E)]
        m1 = v[0]
        for e in range(1, _E):
            m1 = jnp.maximum(m1, v[e])
        sent = jnp.full((_LANES,), _E, jnp.int32)
        idx1 = sent
        for e in range(_E):
            ev = jnp.full((_LANES,), e, jnp.int32)
            take = (v[e] == m1) & (idx1 == sent)
            idx1 = jnp.where(take, ev, idx1)
        neg = jnp.full((_LANES,), -jnp.inf, jnp.float32)
        m2 = neg
        for e in range(_E):
            ev = jnp.full((_LANES,), e, jnp.int32)
            m2 = jnp.maximum(m2, jnp.where(idx1 == ev, neg, v[e]))
        idx2 = sent
        for e in range(_E):
            ev = jnp.full((_LANES,), e, jnp.int32)
            take = (v[e] == m2) & (idx2 == sent) & (idx1 != ev)
            idx2 = jnp.where(take, ev, idx2)
        r2 = jnp.exp(m2 - m1)
        tot = r2 + 1.0
        w1v = 1.0 / tot
        w2v = r2 / tot
        zero = jnp.zeros((_LANES,), jnp.float32)
        for e in range(_E):
            ev = jnp.full((_LANES,), e, jnp.int32)
            ot[pl.ds(e * _LANES, _LANES)] = jnp.where(
                idx1 == ev, w1v, jnp.where(idx2 == ev, w2v, zero))
        for e in range(_E):
            pltpu.sync_copy(ot.at[pl.ds(e * _LANES, _LANES)],
                            out_hbm.at[e, 0, pl.ds(wid * _LANES, _LANES)])


def _route_sc(logits_chunks):
    mesh = plsc.VectorSubcoreMesh(core_axis_name="c", subcore_axis_name="s",
                                  num_cores=1)
    k = functools.partial(
        pl.kernel,
        mesh=mesh,
        out_type=jax.ShapeDtypeStruct((_E, 1, _T), jnp.float32),
        scratch_types=[
            pltpu.VMEM((_E * _LANES,), jnp.float32),
            pltpu.VMEM((_E * _LANES,), jnp.float32),
        ],
    )(_route_sc_body)
    return k(logits_chunks)


def _moe_tc_body(x_ref, rt_ref, w1_ref, w3_ref, w2_ref, out_ref):
    e = pl.program_id(0)
    f = pl.program_id(1)

    @pl.when((e == 0) & (f == 0))
    def _():
        out_ref[...] = jnp.zeros_like(out_ref)

    xb = x_ref[...]
    dn = (((1,), (1,)), ((), ()))
    h1 = lax.dot_general(xb, w1_ref[0], dn, preferred_element_type=jnp.float32)
    h3 = lax.dot_general(xb, w3_ref[0], dn, preferred_element_type=jnp.float32)
    h = h1 * lax.logistic(h1) * h3
    eye = (lax.broadcasted_iota(jnp.int32, (_T, _T), 0)
           == lax.broadcasted_iota(jnp.int32, (_T, _T), 1)).astype(jnp.float32)
    scale = lax.dot_general(eye, rt_ref[0], dn,
                            preferred_element_type=jnp.float32)
    h = h * scale
    out_ref[...] += lax.dot_general(h, w2_ref[0], dn,
                                    preferred_element_type=jnp.float32)


def _moe_tc(x, route_exp, w1_weight, w3_weight, w2_weight):
    grid = (_E, _F // _BF)
    return pl.pallas_call(
        _moe_tc_body,
        grid=grid,
        in_specs=[
            pl.BlockSpec((_T, _D), lambda e, f: (0, 0)),
            pl.BlockSpec((1, 1, _T), lambda e, f: (e, 0, 0)),
            pl.BlockSpec((1, _BF, _D), lambda e, f: (e, f, 0)),
            pl.BlockSpec((1, _BF, _D), lambda e, f: (e, f, 0)),
            pl.BlockSpec((1, _D, _BF), lambda e, f: (e, 0, f)),
        ],
        out_specs=pl.BlockSpec((_T, _D), lambda e, f: (0, 0)),
        out_shape=jax.ShapeDtypeStruct((_T, _D), jnp.float32),
        compiler_params=pltpu.CompilerParams(
            dimension_semantics=("arbitrary", "arbitrary"),
        ),
    )(x, route_exp, w1_weight, w3_weight, w2_weight)


def kernel(x, router_logits, w1_weight, w3_weight, w2_weight):
    nchunks = _T // _LANES
    lg = (router_logits.astype(jnp.float32)
          .reshape(nchunks, _LANES, _E).transpose(0, 2, 1).reshape(nchunks, -1))
    route = _route_sc(lg)
    return _moe_tc(x, route, w1_weight, w3_weight, w2_weight)


# speedup vs baseline: 1.1409x; 1.0002x over previous
import functools

import jax
import jax.numpy as jnp
from jax import lax
from jax.experimental import pallas as pl
from jax.experimental.pallas import tpu as pltpu
from jax.experimental.pallas import tpu_sc as plsc

_T, _D, _F, _E = 64, 2048, 4096, 8
_LANES = 16
_BF = 512


def _route_sc_body(lg_hbm, out_hbm, lt, ot):
    wid = lax.axis_index("s")

    @pl.when(wid < _T // _LANES)
    def _():
        pltpu.sync_copy(lg_hbm.at[wid], lt)
        v = [lt[pl.ds(e * _LANES, _LANES)] for e in range(_E)]
        m1 = v[0]
        for e in range(1, _E):
            m1 = jnp.maximum(m1, v[e])
        sent = jnp.full((_LANES,), _E, jnp.int32)
        idx1 = sent
        for e in range(_E):
            ev = jnp.full((_LANES,), e, jnp.int32)
            take = (v[e] == m1) & (idx1 == sent)
            idx1 = jnp.where(take, ev, idx1)
        neg = jnp.full((_LANES,), -jnp.inf, jnp.float32)
        m2 = neg
        for e in range(_E):
            ev = jnp.full((_LANES,), e, jnp.int32)
            m2 = jnp.maximum(m2, jnp.where(idx1 == ev, neg, v[e]))
        idx2 = sent
        for e in range(_E):
            ev = jnp.full((_LANES,), e, jnp.int32)
            take = (v[e] == m2) & (idx2 == sent) & (idx1 != ev)
            idx2 = jnp.where(take, ev, idx2)
        r2 = jnp.exp(m2 - m1)
        tot = r2 + 1.0
        w1v = 1.0 / tot
        w2v = r2 / tot
        zero = jnp.zeros((_LANES,), jnp.float32)
        for e in range(_E):
            ev = jnp.full((_LANES,), e, jnp.int32)
            ot[pl.ds(e * _LANES, _LANES)] = jnp.where(
                idx1 == ev, w1v, jnp.where(idx2 == ev, w2v, zero))
        for e in range(_E):
            pltpu.sync_copy(ot.at[pl.ds(e * _LANES, _LANES)],
                            out_hbm.at[e, 0, pl.ds(wid * _LANES, _LANES)])


def _route_sc(logits_chunks):
    mesh = plsc.VectorSubcoreMesh(core_axis_name="c", subcore_axis_name="s",
                                  num_cores=1)
    k = functools.partial(
        pl.kernel,
        mesh=mesh,
        out_type=jax.ShapeDtypeStruct((_E, 1, _T), jnp.float32),
        scratch_types=[
            pltpu.VMEM((_E * _LANES,), jnp.float32),
            pltpu.VMEM((_E * _LANES,), jnp.float32),
        ],
    )(_route_sc_body)
    return k(logits_chunks)


def _moe_tc_body(x_ref, w1_ref, w3_ref, w2_ref, out_ref):
    f = pl.program_id(1)

    @pl.when(f == 0)
    def _():
        out_ref[...] = jnp.zeros_like(out_ref)

    xb = x_ref[...]
    dn = (((1,), (1,)), ((), ()))
    h1 = lax.dot_general(xb, w1_ref[0], dn, preferred_element_type=jnp.float32)
    h3 = lax.dot_general(xb, w3_ref[0], dn, preferred_element_type=jnp.float32)
    h = h1 * lax.logistic(h1) * h3
    out_ref[0] += lax.dot_general(h, w2_ref[0], dn,
                                  preferred_element_type=jnp.float32)


def _moe_tc(x, w1_weight, w3_weight, w2_weight):
    grid = (_E, _F // _BF)
    return pl.pallas_call(
        _moe_tc_body,
        grid=grid,
        in_specs=[
            pl.BlockSpec((_T, _D), lambda e, f: (0, 0)),
            pl.BlockSpec((1, _BF, _D), lambda e, f: (e, f, 0)),
            pl.BlockSpec((1, _BF, _D), lambda e, f: (e, f, 0)),
            pl.BlockSpec((1, _D, _BF), lambda e, f: (e, 0, f)),
        ],
        out_specs=pl.BlockSpec((1, _T, _D), lambda e, f: (e, 0, 0)),
        out_shape=jax.ShapeDtypeStruct((_E, _T, _D), jnp.float32),
        compiler_params=pltpu.CompilerParams(
            dimension_semantics=("arbitrary", "arbitrary"),
        ),
    )(x, w1_weight, w3_weight, w2_weight)


def _combine_body(pe_ref, rt_ref, out_ref):
    dn = (((1,), (1,)), ((), ()))
    eye = (lax.broadcasted_iota(jnp.int32, (_T, _T), 0)
           == lax.broadcasted_iota(jnp.int32, (_T, _T), 1)).astype(jnp.float32)
    acc = jnp.zeros((_T, _D), jnp.float32)
    for e in range(_E):
        scale = lax.dot_general(eye, rt_ref[e], dn,
                                preferred_element_type=jnp.float32)
        acc = acc + pe_ref[e] * scale
    out_ref[...] = acc


def _combine_tc(per_expert, route):
    return pl.pallas_call(
        _combine_body,
        out_shape=jax.ShapeDtypeStruct((_T, _D), jnp.float32),
    )(per_expert, route)


def kernel(x, router_logits, w1_weight, w3_weight, w2_weight):
    nchunks = _T // _LANES
    lg = (router_logits.astype(jnp.float32)
          .reshape(nchunks, _LANES, _E).transpose(0, 2, 1).reshape(nchunks, -1))
    route = _route_sc(lg)
    per_expert = _moe_tc(x, w1_weight, w3_weight, w2_weight)
    return _combine_tc(per_expert, route)
